# baseline (speedup 1.0000x reference)
import functools

import jax
import jax.numpy as jnp
from jax import lax
from jax.experimental import pallas as pl
from jax.experimental.pallas import tpu as pltpu
from jax.experimental.pallas import tpu_sc as plsc

NA = 1_600_000
NG = 100_000
LANES = 128
ABLK = 12_800
NBLK = NA // ABLK
SB = 2_560
NSB = ABLK // SB
NSBG = NA // SB
NW = 32
BPW = 4
RGT = ABLK * 4 + 64
RG_TRASH = ABLK * 4
GS = 3_136
NG_PAD = NW * GS
ACC_T = GS * 8


@functools.lru_cache(maxsize=None)
def _mesh():
    return plsc.VectorSubcoreMesh(core_axis_name="c", subcore_axis_name="s")


def _i16():
    return lax.iota(jnp.int32, 16)


def _worker_id():
    return lax.axis_index("s") * 2 + lax.axis_index("c")


@functools.lru_cache(maxsize=None)
def _make_a():
    return pl.kernel(
        _a,
        mesh=_mesh(),
        compiler_params=pltpu.CompilerParams(needs_layout_passes=False),
        out_type=jax.ShapeDtypeStruct((NBLK, 3, ABLK), jnp.float32),
        scratch_types=[
            pltpu.VMEM((SB + 16,), jnp.int32),
            pltpu.VMEM((SB * 9,), jnp.float32),
            pltpu.VMEM((SB * 9,), jnp.float32),
            pltpu.VMEM(((SB + 40) * 3,), jnp.float32),
            pltpu.VMEM((RGT,), jnp.float32),
            pltpu.VMEM((3, SB), jnp.float32),
            pltpu.VMEM((32,), jnp.int32),
            pltpu.VMEM((32,), jnp.float32),
            pltpu.SemaphoreType.DMA,
        ],
    )


def _a(ids_hbm, rot9_hbm, inv9_hbm, bnp_hbm, rx_hbm,
       ids_v, rot_v, inv_v, bn_v, rg_v, ob_v, sw_v, ww_v, sem):
    w = _worker_id()
    i16 = _i16()
    b_lo = w * BPW
    b_hi = jnp.minimum(b_lo + BPW, NBLK)
    base0 = pl.multiple_of(b_lo * ABLK, 128)

    pltpu.async_copy(ids_hbm.at[pl.ds(base0, 16)], sw_v.at[pl.ds(0, 16)],
                     sem).wait()
    g_head0 = sw_v[pl.ds(0, 16)][0]

    def srch(t, carry):
        lo, hi = carry
        mid = (lo + hi) // 2
        pltpu.async_copy(
            ids_hbm.at[pl.ds(pl.multiple_of(mid * 16, 16), 16)],
            sw_v.at[pl.ds(16, 16)], sem).wait()
        vmax = sw_v[pl.ds(16, 16)][15]
        found = vmax >= g_head0
        return (jnp.where(found, lo, mid + 1), jnp.where(found, mid, hi))
    glo, _ = lax.fori_loop(0, 17, srch, (jnp.int32(0), base0 // 16))
    pltpu.async_copy(
        ids_hbm.at[pl.ds(pl.multiple_of(glo * 16, 16), 16)],
        sw_v.at[pl.ds(16, 16)], sem).wait()
    vg = sw_v[pl.ds(16, 16)]
    a_star = glo * 16 + plsc.all_reduce_ffs(vg >= g_head0)[0]

    a9 = a_star * 9
    w0 = pl.multiple_of((a9 // 8) * 8, 8)
    pltpu.async_copy(inv9_hbm.at[pl.ds(w0, 24)], ww_v.at[pl.ds(0, 24)],
                     sem).wait()
    iv = plsc.load_gather(ww_v, [jnp.minimum(a9 - w0 + i16, 23)])
    g3 = g_head0 * 3
    w1 = pl.multiple_of((g3 // 8) * 8, 8)
    pltpu.async_copy(bnp_hbm.at[pl.ds(w1, 16)], ww_v.at[pl.ds(0, 16)],
                     sem).wait()
    bv = plsc.load_gather(ww_v, [jnp.minimum(g3 - w1 + i16, 15)])

    hr = []
    for i in range(3):
        acc = jnp.zeros((), jnp.float32)
        for j in range(3):
            acc = acc + iv[3 * i + j] * bv[j]
        hr.append(acc)
    head_rg = jnp.where(i16 == 0, hr[0],
                        jnp.where(i16 == 1, hr[1],
                                  jnp.where(i16 == 2, hr[2], 0.0)))
    plsc.store_scatter(rg_v, [jnp.where(i16 < 3, i16, RG_TRASH)], head_rg)

    def block_body(bb, g_carry):
        b = b_lo + bb
        base = pl.multiple_of(b * ABLK, 128)
        g_lo_prev = g_carry

        def sb_body(s, g_lo_blk):
            o = s * SB
            a0 = pl.multiple_of(base + o, 128)

            first = (s == 0) & (b == b_lo) & (base0 == 0)

            @pl.when(jnp.logical_not(first))
            def _():
                pltpu.async_copy(
                    ids_hbm.at[pl.ds(pl.multiple_of(a0 - 16, 16), SB + 16)],
                    ids_v, sem).wait()

            @pl.when(first)
            def _():
                pltpu.async_copy(ids_hbm.at[pl.ds(a0, SB)],
                                 ids_v.at[pl.ds(16, SB)], sem).wait()
                ids_v[pl.ds(0, 16)] = jnp.full((16,), -1, jnp.int32)

            g_sb = ids_v[pl.ds(16, 16)][0]
            gb3 = pl.multiple_of(((g_sb * 3) // 8) * 8, 8)
            pltpu.async_copy(
                rot9_hbm.at[pl.ds(pl.multiple_of(a0 * 9, 128), SB * 9)],
                rot_v, sem).wait()
            pltpu.async_copy(
                inv9_hbm.at[pl.ds(pl.multiple_of(a0 * 9, 128), SB * 9)],
                inv_v, sem).wait()
            pltpu.async_copy(bnp_hbm.at[pl.ds(gb3, (SB + 40) * 3)],
                             bn_v, sem).wait()

            def grp_body(k, _):
                cur = ids_v[pl.ds(16 + k * 16, 16)]
                prv = ids_v[pl.ds(15 + k * 16, 16)]
                bm = cur != prv
                gl = cur - g_lo_blk
                la = jnp.full((16,), k * 16, jnp.int32) + i16
                la9 = la * 9
                bo = cur * 3 - gb3
                bc = [plsc.load_gather(bn_v, [bo + jj]) for jj in range(3)]
                gl4 = gl * 4
                for i in range(3):
                    acc = jnp.zeros((16,), jnp.float32)
                    for jj in range(3):
                        ic = plsc.load_gather(inv_v, [la9 + (3 * i + jj)])
                        acc = acc + ic * bc[jj]
                    dst = jnp.where(bm, gl4 + i, jnp.int32(RG_TRASH))
                    plsc.store_scatter(rg_v, [dst], acc)
                rc = [plsc.load_gather(rg_v, [gl4 + jj]) for jj in range(3)]
                for i in range(3):
                    acc = jnp.zeros((16,), jnp.float32)
                    for jj in range(3):
                        mc = plsc.load_gather(rot_v, [la9 + (3 * i + jj)])
                        acc = acc + mc * rc[jj]
                    ob_v[i, pl.ds(k * 16, 16)] = acc
                return 0
            lax.fori_loop(0, SB // 16, grp_body, 0)
            pltpu.async_copy(ob_v, rx_hbm.at[b, :, pl.ds(o, SB)], sem).wait()
            return g_lo_blk
        lax.fori_loop(0, NSB, sb_body, g_lo_prev)

        nb = pl.multiple_of(base + ABLK, 128)

        @pl.when(bb + 1 < b_hi - b_lo)
        def _():
            pltpu.async_copy(ids_hbm.at[pl.ds(nb, 16)],
                             sw_v.at[pl.ds(0, 16)], sem).wait()

        g_next = jnp.where(bb + 1 < b_hi - b_lo, sw_v[pl.ds(0, 16)][0],
                           g_lo_prev)
        src = jnp.minimum((g_next - g_lo_prev) * 4, RG_TRASH) \
            + jnp.where(i16 < 4, i16, 0)
        row = plsc.load_gather(rg_v, [src])
        plsc.store_scatter(rg_v, [jnp.where(i16 < 4, i16, RG_TRASH)], row)
        return g_next
    lax.fori_loop(0, b_hi - b_lo, block_body, g_head0)


def _b_body(rx_ref, sig_ref, nt_ref, q_ref):
    i = pl.program_id(0)
    sig = sig_ref[0] + jnp.float32(1e-2)
    var = sig * sig
    inv_var = 1.0 / var
    n2v = -0.5 * inv_var
    inv_rs = 1.0 / jnp.sqrt(sig)
    qacc = jnp.zeros((), jnp.float32)
    for d in range(3):
        x = sig * rx_ref[0, d]
        nt_ref[0, d] = x
        p_ = jnp.zeros_like(x)
        s_ = jnp.zeros_like(x)
        for k in range(-10, 11):
            xi = x + jnp.float32(k)
            e = jnp.exp(xi * xi * n2v)
            p_ = p_ + xi * inv_var * e
            s_ = s_ + e
        t = p_ / (s_ + jnp.float32(1e-12)) * inv_rs
        nt_ref[0, 3 + d] = t
        qacc = qacc + jnp.sum(t * t)

    @pl.when(i == 0)
    def _():
        q_ref[0, 0] = jnp.float32(0.0)
    q_ref[0, 0] += qacc


def _run_b(rx4, sig3d):
    return pl.pallas_call(
        _b_body,
        grid=(NBLK,),
        in_specs=[
            pl.BlockSpec((1, 3, ABLK // LANES, LANES), lambda i: (i, 0, 0, 0)),
            pl.BlockSpec((1, ABLK // LANES, LANES), lambda i: (i, 0, 0)),
        ],
        out_specs=[
            pl.BlockSpec((1, 6, ABLK // LANES, LANES), lambda i: (i, 0, 0, 0)),
            pl.BlockSpec(memory_space=pltpu.SMEM),
        ],
        out_shape=[
            jax.ShapeDtypeStruct((NBLK, 6, ABLK // LANES, LANES), jnp.float32),
            jax.ShapeDtypeStruct((1, 1), jnp.float32),
        ],
    )(rx4, sig3d)


@functools.lru_cache(maxsize=None)
def _make_c():
    return pl.kernel(
        _c,
        mesh=_mesh(),
        compiler_params=pltpu.CompilerParams(needs_layout_passes=False),
        out_type=jax.ShapeDtypeStruct((NG_PAD * 8,), jnp.float32),
        scratch_types=[
            pltpu.VMEM((SB,), jnp.int32),
            pltpu.VMEM((SB * 3,), jnp.float32),
            pltpu.VMEM((6, SB), jnp.float32),
            pltpu.VMEM((ACC_T + 16,), jnp.float32),
            pltpu.VMEM((32,), jnp.int32),
            pltpu.SemaphoreType.DMA,
        ],
    )


def _c(ids_hbm, frac_hbm, nt_hbm, acc_hbm,
       ids_v, frac_v, nt_v, acc_v, sw_v, sem):
    w = _worker_id()
    i16 = _i16()
    s_lo = w * GS
    s_hi = s_lo + GS

    def z_body(k, _):
        acc_v[pl.ds(k * 16, 16)] = jnp.zeros((16,), jnp.float32)
        return 0
    lax.fori_loop(0, (ACC_T + 16) // 16, z_body, 0)

    def srch(val, off):
        def step(t, carry):
            lo, hi = carry
            mid = (lo + hi) // 2
            pltpu.async_copy(
                ids_hbm.at[pl.ds(pl.multiple_of(mid * SB + off, 16), 16)],
                sw_v.at[pl.ds(0, 16)], sem).wait()
            v = sw_v[pl.ds(0, 16)][15 if off else 0]
            found = v >= val
            return (jnp.where(found, lo, mid + 1), jnp.where(found, mid, hi))
        lo, _ = lax.fori_loop(0, 11, step, (jnp.int32(0), jnp.int32(NSBG)))
        return lo
    sb_lo = srch(s_lo, SB - 16)
    sb_hi = srch(s_hi, 0)

    def sb_body(t, _):
        sbg = sb_lo + t
        b = sbg // NSB
        o = (sbg % NSB) * SB
        a0 = pl.multiple_of(sbg * SB, 128)
        pltpu.async_copy(ids_hbm.at[pl.ds(a0, SB)], ids_v, sem).wait()
        pltpu.async_copy(
            frac_hbm.at[pl.ds(pl.multiple_of(a0 * 3, 128), SB * 3)],
            frac_v, sem).wait()
        pltpu.async_copy(nt_hbm.at[b, :, pl.ds(o, SB)], nt_v, sem).wait()

        def grp_body(k, _):
            rr = k * 16 + i16
            ids16 = ids_v[pl.ds(k * 16, 16)]
            inr = (ids16 >= s_lo) & (ids16 < s_hi)
            b8 = jnp.where(inr, (ids16 - s_lo) * 8, jnp.int32(ACC_T))
            r3 = rr * 3
            for d in range(3):
                nz = nt_v[d, pl.ds(k * 16, 16)]
                fc = plsc.load_gather(frac_v, [r3 + d])
                x = fc + nz
                xt = x.astype(jnp.int32).astype(jnp.float32)
                fl = x - xt
                fl = jnp.where(fl < 0, fl + 1.0, fl)
                plsc.addupdate_scatter(acc_v, [b8 + d], fl)
                tv = nt_v[3 + d, pl.ds(k * 16, 16)]
                plsc.addupdate_scatter(acc_v, [b8 + (3 + d)], tv)
            plsc.addupdate_scatter(acc_v, [b8 + 6],
                                   jnp.full((16,), 1.0, jnp.float32))
            return 0
        lax.fori_loop(0, SB // 16, grp_body, 0)
        return 0
    lax.fori_loop(0, jnp.maximum(sb_hi - sb_lo, 0), sb_body, 0)

    pltpu.sync_copy(acc_v.at[pl.ds(0, ACC_T)],
                    acc_hbm.at[pl.ds(pl.multiple_of(w * ACC_T, 128), ACC_T)])


def _d_body(acc_ref, q_ref, out_ref):
    x = acc_ref[...]
    lane = lax.broadcasted_iota(jnp.int32, x.shape, 1) % 8
    tb = jnp.roll(x, -3, axis=1)
    c6 = jnp.roll(x, -6, axis=1)
    c5 = jnp.roll(x, -5, axis=1)
    c4 = jnp.roll(x, -4, axis=1)
    cb = jnp.where(lane == 0, c6, jnp.where(lane == 1, c5, c4))
    cp = jnp.maximum(cb, 1.0)
    term = (x * x) * cb / (cp * cp) - 2.0 * x * tb / cp
    term = jnp.where(lane < 3, term, 0.0)
    out_ref[0, 0] = (jnp.sum(term) + q_ref[0, 0]) * jnp.float32(1.0 / (3.0 * NA))


def _run_d(acc3, q):
    return pl.pallas_call(
        _d_body,
        in_specs=[
            pl.BlockSpec((NG_PAD * 8 // LANES, LANES), lambda: (0, 0)),
            pl.BlockSpec(memory_space=pltpu.SMEM),
        ],
        out_specs=pl.BlockSpec(memory_space=pltpu.SMEM),
        out_shape=jax.ShapeDtypeStruct((1, 1), jnp.float32),
    )(acc3, q)


def kernel(frac_coords, rotation, inv_rotation, base_noise, sigmas_per_atom,
           segment_ids):
    ids = segment_ids.astype(jnp.int32)
    inv9 = inv_rotation.reshape(NA * 9)
    rot9 = rotation.reshape(NA * 9)
    bnp = jnp.concatenate(
        [base_noise, jnp.zeros((SB + 80, 3), jnp.float32)], axis=0).reshape(-1)
    sig3d = sigmas_per_atom.reshape(NBLK, ABLK // LANES, LANES)
    fracf = frac_coords.reshape(NA * 3)

    rx = _make_a()(ids, rot9, inv9, bnp)
    rx4 = rx.reshape(NBLK, 3, ABLK // LANES, LANES)
    nt, q = _run_b(rx4, sig3d)
    ntv = nt.reshape(NBLK, 6, ABLK)
    acc = _make_c()(ids, fracf, ntv)
    acc3 = acc.reshape(NG_PAD * 8 // LANES, LANES)
    loss = _run_d(acc3, q)
    return loss.reshape(())
